# baseline (speedup 1.0000x reference)
import jax
import jax.numpy as jnp
from jax.experimental import pallas as pl
from jax.experimental.pallas import tpu as pltpu


def _build_idx_list():
    num_candidates = 16
    indices = [0, 1, 2, 3, 4, 5, 6, 7, 8]
    base_idx = 9
    for i in range(num_candidates - 1):
        indices += [6, 7, base_idx + i]
    indices += [0, 3, 6, 1, 4, 7, 2, 5, 8]
    for i in range(num_candidates - 1):
        indices += [2, 5, base_idx + i]
    return indices


_IDX = _build_idx_list()


def _merge_runs(idx):
    runs = []
    o_start, s_start, length = 0, idx[0], 1
    for j in range(1, len(idx)):
        if idx[j] == s_start + length:
            length += 1
        else:
            runs.append((o_start, s_start, length))
            o_start, s_start, length = j, idx[j], 1
    runs.append((o_start, s_start, length))
    return runs


_RUNS = _merge_runs(_IDX)


_BPB = 4


def _body(x_ref, o_hbm, sem):
    i = pl.program_id(0)
    copies = [
        pltpu.make_async_copy(
            x_ref.at[bb, pl.ds(s_start, length)],
            o_hbm.at[i * _BPB + bb, pl.ds(o_start, length)],
            sem,
        )
        for bb in range(_BPB)
        for o_start, s_start, length in _RUNS
    ]
    for c in copies:
        c.start()
    for c in copies:
        c.wait()


def kernel(x):
    b, n, s, d = x.shape
    n_out = len(_IDX)

    out = pl.pallas_call(
        _body,
        grid=(b // _BPB,),
        in_specs=[pl.BlockSpec((_BPB, n, s, d), lambda i: (i, 0, 0, 0))],
        out_specs=pl.BlockSpec(memory_space=pl.ANY),
        out_shape=jax.ShapeDtypeStruct((b, n_out, s, d), x.dtype),
        scratch_shapes=[pltpu.SemaphoreType.DMA],
    )(x)
    return out.reshape(b, n_out // 3, 3, s, d)


# speedup vs baseline: 1.0081x; 1.0081x over previous
import jax
import jax.numpy as jnp
from jax.experimental import pallas as pl
from jax.experimental.pallas import tpu as pltpu


def _build_idx_list():
    num_candidates = 16
    indices = [0, 1, 2, 3, 4, 5, 6, 7, 8]
    base_idx = 9
    for i in range(num_candidates - 1):
        indices += [6, 7, base_idx + i]
    indices += [0, 3, 6, 1, 4, 7, 2, 5, 8]
    for i in range(num_candidates - 1):
        indices += [2, 5, base_idx + i]
    return indices


_IDX = _build_idx_list()


def _merge_runs(idx):
    runs = []
    o_start, s_start, length = 0, idx[0], 1
    for j in range(1, len(idx)):
        if idx[j] == s_start + length:
            length += 1
        else:
            runs.append((o_start, s_start, length))
            o_start, s_start, length = j, idx[j], 1
    runs.append((o_start, s_start, length))
    return runs


_RUNS = _merge_runs(_IDX)


_BPB = 2


def _body(x_ref, o_hbm, sem):
    i = pl.program_id(0)
    copies = [
        pltpu.make_async_copy(
            x_ref.at[bb, pl.ds(s_start, length)],
            o_hbm.at[i * _BPB + bb, pl.ds(o_start, length)],
            sem,
        )
        for bb in range(_BPB)
        for o_start, s_start, length in _RUNS
    ]
    for c in copies:
        c.start()
    for c in copies:
        c.wait()


def kernel(x):
    b, n, s, d = x.shape
    n_out = len(_IDX)

    out = pl.pallas_call(
        _body,
        grid=(b // _BPB,),
        in_specs=[pl.BlockSpec((_BPB, n, s, d), lambda i: (i, 0, 0, 0))],
        out_specs=pl.BlockSpec(memory_space=pl.ANY),
        out_shape=jax.ShapeDtypeStruct((b, n_out, s, d), x.dtype),
        scratch_shapes=[pltpu.SemaphoreType.DMA],
    )(x)
    return out.reshape(b, n_out // 3, 3, s, d)
